# baseline (speedup 1.0000x reference)
import jax
import jax.numpy as jnp
from jax.experimental import pallas as pl
from jax.experimental.pallas import tpu as pltpu

Q = 16
C = 100
P = 196
D = 768
PB = 32
NFULL = P // PB
TAIL = P - NFULL * PB
QR = Q * PB
CR = C * PB


def _body(q_ref, k_ref, dist_ref, idx_ref, g8_ref, s2_ref, q2_ref):
    pp = pl.program_id(0)

    @pl.when(pp == 0)
    def _init():
        g8_ref[...] = jnp.zeros_like(g8_ref)
        s2_ref[...] = jnp.zeros_like(s2_ref)
        q2_ref[...] = jnp.zeros_like(q2_ref)

    @pl.when(pp < NFULL)
    def _main():
        qblk = q_ref[...]
        kblk = k_ref[...]
        qr = qblk.reshape(QR, D)
        kr = kblk.reshape(CR, D)
        qhi = qr.astype(jnp.bfloat16)
        qlo = (qr - qhi.astype(jnp.float32)).astype(jnp.bfloat16)
        khi = kr.astype(jnp.bfloat16)
        klo = (kr - khi.astype(jnp.float32)).astype(jnp.bfloat16)

        def _dot(a, b):
            return jax.lax.dot_general(
                a, b, (((1,), (1,)), ((), ())),
                preferred_element_type=jnp.float32,
            )

        g8_ref[...] += _dot(qhi, khi) + _dot(qhi, klo) + _dot(qlo, khi)
        s2_ref[...] += kblk * kblk
        q2_ref[...] += qblk * qblk

    @pl.when(pp == NFULL)
    def _fin():
        dtail = jnp.zeros((Q, C), jnp.float32)
        s2t = jnp.zeros((C, D), jnp.float32)
        q2t = jnp.zeros((Q, D), jnp.float32)
        for p in range(TAIL):
            qp = q_ref[p, :, :]
            kp = k_ref[:, p, :]
            dtail += jax.lax.dot_general(
                qp, kp, (((1,), (1,)), ((), ())),
                preferred_element_type=jnp.float32,
                precision=jax.lax.Precision.HIGHEST,
            )
            s2t += kp * kp
            q2t += qp * qp

        g8 = g8_ref[...]
        row = jax.lax.broadcasted_iota(jnp.int32, (QR, CR), 0)
        col = jax.lax.broadcasted_iota(jnp.int32, (QR, CR), 1)
        g8m = jnp.where((row // Q) == (col % PB), g8, 0.0)
        srow = jax.lax.broadcasted_iota(jnp.int32, (Q, QR), 0)
        scol = jax.lax.broadcasted_iota(jnp.int32, (Q, QR), 1)
        s_fold = jnp.where(srow == scol % Q, 1.0, 0.0)
        frow = jax.lax.broadcasted_iota(jnp.int32, (CR, C), 0)
        fcol = jax.lax.broadcasted_iota(jnp.int32, (CR, C), 1)
        f_fold = jnp.where(frow // PB == fcol, 1.0, 0.0)
        gq = jax.lax.dot_general(
            s_fold, g8m, (((1,), (0,)), ((), ())),
            preferred_element_type=jnp.float32,
            precision=jax.lax.Precision.HIGHEST,
        )
        dot = jax.lax.dot_general(
            gq, f_fold, (((1,), (0,)), ((), ())),
            preferred_element_type=jnp.float32,
            precision=jax.lax.Precision.HIGHEST,
        ) + dtail

        ks = jnp.sum(s2_ref[...], axis=(1, 2)) + jnp.sum(s2t, axis=1)
        qs = (jnp.sum(q2_ref[...], axis=(0, 2))
              + jnp.sum(q2t, axis=1))[:, None]
        d2 = jnp.maximum(qs + ks[None, :] - 2.0 * dot, 0.0)
        idx_ref[...] = jnp.argmin(d2, axis=1).astype(jnp.int32)
        dist_ref[...] = jnp.sqrt(jnp.min(d2, axis=1))


def kernel(query_features, keys):
    qt = jnp.swapaxes(query_features, 0, 1)
    dist, idx = pl.pallas_call(
        _body,
        grid=(NFULL + 1,),
        in_specs=[
            pl.BlockSpec((PB, Q, D), lambda p: (p, 0, 0)),
            pl.BlockSpec((C, PB, D), lambda p: (0, p, 0)),
        ],
        out_specs=[
            pl.BlockSpec((Q,), lambda p: (0,)),
            pl.BlockSpec((Q,), lambda p: (0,)),
        ],
        out_shape=[
            jax.ShapeDtypeStruct((Q,), jnp.float32),
            jax.ShapeDtypeStruct((Q,), jnp.int32),
        ],
        scratch_shapes=[
            pltpu.VMEM((QR, CR), jnp.float32),
            pltpu.VMEM((C, PB, D), jnp.float32),
            pltpu.VMEM((PB, Q, D), jnp.float32),
        ],
    )(qt, keys)
    return dist, idx


# speedup vs baseline: 1.0549x; 1.0549x over previous
import functools

import jax
import jax.numpy as jnp
from jax.experimental import pallas as pl
from jax.experimental.pallas import tpu as pltpu
from jax.experimental.pallas import tpu_sc as plsc

Q = 16
C = 100
P = 196
D = 768
PB = 16
NFULL = P // PB
TAIL = P - NFULL * PB
QR = Q * PB
CR = C * PB


def _body(q_ref, k_ref, dist_ref, g8_ref, s2_ref, q2_ref):
    pp = pl.program_id(0)

    @pl.when(pp == 0)
    def _init():
        g8_ref[...] = jnp.zeros_like(g8_ref)
        s2_ref[...] = jnp.zeros_like(s2_ref)
        q2_ref[...] = jnp.zeros_like(q2_ref)

    @pl.when(pp < NFULL)
    def _main():
        qblk = q_ref[...]
        kblk = k_ref[...]
        qr = qblk.reshape(QR, D)
        kr = kblk.reshape(CR, D)
        qhi = qr.astype(jnp.bfloat16)
        qlo = (qr - qhi.astype(jnp.float32)).astype(jnp.bfloat16)
        khi = kr.astype(jnp.bfloat16)
        klo = (kr - khi.astype(jnp.float32)).astype(jnp.bfloat16)

        def _dot(a, b):
            return jax.lax.dot_general(
                a, b, (((1,), (1,)), ((), ())),
                preferred_element_type=jnp.float32,
            )

        g8_ref[...] += _dot(qhi, khi) + _dot(qhi, klo) + _dot(qlo, khi)
        s2_ref[...] += kblk * kblk
        q2_ref[...] += qblk * qblk

    @pl.when(pp == NFULL)
    def _fin():
        dtail = jnp.zeros((Q, C), jnp.float32)
        s2t = jnp.zeros((C, D), jnp.float32)
        q2t = jnp.zeros((Q, D), jnp.float32)
        for p in range(TAIL):
            qp = q_ref[p, :, :]
            kp = k_ref[:, p, :]
            dtail += jax.lax.dot_general(
                qp, kp, (((1,), (1,)), ((), ())),
                preferred_element_type=jnp.float32,
                precision=jax.lax.Precision.HIGHEST,
            )
            s2t += kp * kp
            q2t += qp * qp

        g8 = g8_ref[...]
        row = jax.lax.broadcasted_iota(jnp.int32, (QR, CR), 0)
        col = jax.lax.broadcasted_iota(jnp.int32, (QR, CR), 1)
        g8m = jnp.where((row // Q) == (col % PB), g8, 0.0)
        srow = jax.lax.broadcasted_iota(jnp.int32, (Q, QR), 0)
        scol = jax.lax.broadcasted_iota(jnp.int32, (Q, QR), 1)
        s_fold = jnp.where(srow == scol % Q, 1.0, 0.0)
        frow = jax.lax.broadcasted_iota(jnp.int32, (CR, C), 0)
        fcol = jax.lax.broadcasted_iota(jnp.int32, (CR, C), 1)
        f_fold = jnp.where(frow // PB == fcol, 1.0, 0.0)
        gq = jax.lax.dot_general(
            s_fold, g8m, (((1,), (0,)), ((), ())),
            preferred_element_type=jnp.float32,
            precision=jax.lax.Precision.HIGHEST,
        )
        dot = jax.lax.dot_general(
            gq, f_fold, (((1,), (0,)), ((), ())),
            preferred_element_type=jnp.float32,
            precision=jax.lax.Precision.HIGHEST,
        ) + dtail

        ks = jnp.sum(s2_ref[...], axis=(1, 2)) + jnp.sum(s2t, axis=1)
        qs = (jnp.sum(q2_ref[...], axis=(0, 2))
              + jnp.sum(q2t, axis=1))[:, None]
        d2 = jnp.maximum(qs + ks[None, :] - 2.0 * dot, 0.0)
        dist = jnp.sqrt(d2)
        dist_ref[...] = jnp.pad(
            dist, ((0, 0), (0, 128 - C)), constant_values=jnp.inf)


def _take16(v, perm):
    return jax.lax.gather(
        v, perm[:, None],
        jax.lax.GatherDimensionNumbers(
            offset_dims=(), collapsed_slice_dims=(0,), start_index_map=(0,)),
        slice_sizes=(1,),
        mode=jax.lax.GatherScatterMode.PROMISE_IN_BOUNDS,
    )


def _sc_select(distpad):
    mesh = plsc.VectorSubcoreMesh(core_axis_name="c", subcore_axis_name="s")

    @functools.partial(
        pl.kernel,
        out_type=[
            jax.ShapeDtypeStruct((Q,), jnp.float32),
            jax.ShapeDtypeStruct((Q,), jnp.int32),
        ],
        mesh=mesh,
        scratch_types=[
            pltpu.VMEM((Q, 128), jnp.float32),
            pltpu.VMEM((Q,), jnp.float32),
            pltpu.VMEM((Q,), jnp.int32),
        ],
    )
    def _sel(d_hbm, dist_hbm, idx_hbm, buf, dv, iv):
        cid = jax.lax.axis_index("c")
        sid = jax.lax.axis_index("s")

        @pl.when(jnp.logical_and(cid == 0, sid == 0))
        def _():
            pltpu.sync_copy(d_hbm, buf)
            ii = jax.lax.iota(jnp.int32, 16)
            dvv = jnp.full((16,), 3.4e38, jnp.float32)
            ivv = jnp.zeros((16,), jnp.int32)
            for i in range(Q):
                minv = jnp.full((16,), 3.4e38, jnp.float32)
                mindi = jnp.zeros((16,), jnp.int32)
                for h in range(128 // 16):
                    v = buf[i, pl.ds(h * 16, 16)]
                    better = v < minv
                    minv = jnp.where(better, v, minv)
                    mindi = jnp.where(better, ii + h * 16, mindi)
                for s in (1, 2, 4, 8):
                    perm = ii ^ s
                    v2 = _take16(minv, perm)
                    i2 = _take16(mindi, perm)
                    better = (v2 < minv) | ((v2 == minv) & (i2 < mindi))
                    minv = jnp.where(better, v2, minv)
                    mindi = jnp.where(better, i2, mindi)
                lane = ii == i
                dvv = jnp.where(lane, minv, dvv)
                ivv = jnp.where(lane, mindi, ivv)
            dv[...] = dvv
            iv[...] = ivv
            pltpu.sync_copy(dv, dist_hbm)
            pltpu.sync_copy(iv, idx_hbm)

    return _sel(distpad)


def kernel(query_features, keys):
    qt = jnp.swapaxes(query_features, 0, 1)
    distpad = pl.pallas_call(
        _body,
        grid=(NFULL + 1,),
        in_specs=[
            pl.BlockSpec((PB, Q, D), lambda p: (p, 0, 0)),
            pl.BlockSpec((C, PB, D), lambda p: (0, p, 0)),
        ],
        out_specs=pl.BlockSpec((Q, 128), lambda p: (0, 0)),
        out_shape=jax.ShapeDtypeStruct((Q, 128), jnp.float32),
        scratch_shapes=[
            pltpu.VMEM((QR, CR), jnp.float32),
            pltpu.VMEM((C, PB, D), jnp.float32),
            pltpu.VMEM((PB, Q, D), jnp.float32),
        ],
    )(qt, keys)
    dist, idx = _sc_select(distpad)
    return dist, idx


# speedup vs baseline: 1.4919x; 1.4143x over previous
import jax
import jax.numpy as jnp
from jax.experimental import pallas as pl
from jax.experimental.pallas import tpu as pltpu

Q = 16
C = 100
P = 196
D = 768
PB = 16
NFULL = P // PB
TAIL = P - NFULL * PB
QR = Q * PB
CR = C * PB


def _body(q_ref, k_ref, dist_ref, idx_ref, g8_ref, s2_ref, q2_ref):
    pp = pl.program_id(0)

    @pl.when(pp == 0)
    def _init():
        g8_ref[...] = jnp.zeros_like(g8_ref)
        s2_ref[...] = jnp.zeros_like(s2_ref)
        q2_ref[...] = jnp.zeros_like(q2_ref)

    @pl.when(pp < NFULL)
    def _main():
        qblk = q_ref[...]
        kblk = k_ref[...]
        qr = qblk.reshape(QR, D)
        kr = kblk.reshape(CR, D)
        qhi = qr.astype(jnp.bfloat16)
        qlo = (qr - qhi.astype(jnp.float32)).astype(jnp.bfloat16)
        khi = kr.astype(jnp.bfloat16)
        klo = (kr - khi.astype(jnp.float32)).astype(jnp.bfloat16)

        def _dot(a, b):
            return jax.lax.dot_general(
                a, b, (((1,), (1,)), ((), ())),
                preferred_element_type=jnp.float32,
            )

        g8_ref[...] += _dot(qhi, khi) + _dot(qhi, klo) + _dot(qlo, khi)
        s2_ref[...] += kblk * kblk
        q2_ref[...] += qblk * qblk

    @pl.when(pp == NFULL)
    def _fin():
        dtail = jnp.zeros((Q, C), jnp.float32)
        s2t = jnp.zeros((C, D), jnp.float32)
        q2t = jnp.zeros((Q, D), jnp.float32)
        for p in range(TAIL):
            qp = q_ref[p, :, :]
            kp = k_ref[:, p, :]
            dtail += jax.lax.dot_general(
                qp, kp, (((1,), (1,)), ((), ())),
                preferred_element_type=jnp.float32,
                precision=jax.lax.Precision.HIGHEST,
            )
            s2t += kp * kp
            q2t += qp * qp

        g8 = g8_ref[...]
        row = jax.lax.broadcasted_iota(jnp.int32, (QR, CR), 0)
        col = jax.lax.broadcasted_iota(jnp.int32, (QR, CR), 1)
        g8m = jnp.where((row // Q) == (col % PB), g8, 0.0)
        srow = jax.lax.broadcasted_iota(jnp.int32, (Q, QR), 0)
        scol = jax.lax.broadcasted_iota(jnp.int32, (Q, QR), 1)
        s_fold = jnp.where(srow == scol % Q, 1.0, 0.0)
        frow = jax.lax.broadcasted_iota(jnp.int32, (CR, C), 0)
        fcol = jax.lax.broadcasted_iota(jnp.int32, (CR, C), 1)
        f_fold = jnp.where(frow // PB == fcol, 1.0, 0.0)
        gq = jax.lax.dot_general(
            s_fold, g8m, (((1,), (0,)), ((), ())),
            preferred_element_type=jnp.float32,
            precision=jax.lax.Precision.HIGHEST,
        )
        dot = jax.lax.dot_general(
            gq, f_fold, (((1,), (0,)), ((), ())),
            preferred_element_type=jnp.float32,
            precision=jax.lax.Precision.HIGHEST,
        ) + dtail

        ks = jnp.sum(s2_ref[...], axis=(1, 2)) + jnp.sum(s2t, axis=1)
        qs = (jnp.sum(q2_ref[...], axis=(0, 2))
              + jnp.sum(q2t, axis=1))[:, None]
        d2 = jnp.maximum(qs + ks[None, :] - 2.0 * dot, 0.0)
        idx_ref[...] = jnp.argmin(d2, axis=1).astype(jnp.int32)
        dist_ref[...] = jnp.sqrt(jnp.min(d2, axis=1))


def kernel(query_features, keys):
    qt = jnp.swapaxes(query_features, 0, 1)
    dist, idx = pl.pallas_call(
        _body,
        grid=(NFULL + 1,),
        in_specs=[
            pl.BlockSpec((PB, Q, D), lambda p: (p, 0, 0)),
            pl.BlockSpec((C, PB, D), lambda p: (0, p, 0)),
        ],
        out_specs=[
            pl.BlockSpec((Q,), lambda p: (0,)),
            pl.BlockSpec((Q,), lambda p: (0,)),
        ],
        out_shape=[
            jax.ShapeDtypeStruct((Q,), jnp.float32),
            jax.ShapeDtypeStruct((Q,), jnp.int32),
        ],
        scratch_shapes=[
            pltpu.VMEM((QR, CR), jnp.float32),
            pltpu.VMEM((C, PB, D), jnp.float32),
            pltpu.VMEM((PB, Q, D), jnp.float32),
        ],
    )(qt, keys)
    return dist, idx
